# Initial kernel scaffold; baseline (speedup 1.0000x reference)
import jax
import jax.numpy as jnp
from jax.experimental import pallas as pl


def kernel(x, edge_index, edge_attr, batch, W1, b1, W2, b2, W3, b3, g1, be1, rm1, rv1, g2, be2, rm2, rv2, g3, be3, rm3, rv3, Wo, bo):
    raise NotImplementedError("write your pallas kernel here")


# speedup vs baseline: 11.0514x; 11.0514x over previous
import functools

import jax
import jax.numpy as jnp
from jax import lax
from jax.experimental import pallas as pl
from jax.experimental.pallas import tpu as pltpu
from jax.experimental.pallas import tpu_sc as plsc

N = 10000
E = 320000
D = 128
H = 128
G = 64
C = 40

NC = 2
NS = 16
NW = NC * NS
EPT = E // NW
K = 80
NCHUNK = EPT // K
RPT = 624
TAIL_OFF = RPT * NS
TAIL = N - TAIL_OFF
BLK = 1000
NBLK = N // BLK

_mesh = functools.partial(
    plsc.VectorSubcoreMesh, core_axis_name="c", subcore_axis_name="s")


def _sc_degree(dst, zeros1, ones1):
    @functools.partial(
        pl.kernel,
        mesh=_mesh(),
        out_type=jax.ShapeDtypeStruct((NC * N,), jnp.float32),
        scratch_types=[
            pltpu.VMEM((K,), jnp.int32),
            pltpu.VMEM((K,), jnp.float32),
            pltpu.VMEM((RPT,), jnp.float32),
            pltpu.VMEM_SHARED((N,), jnp.float32),
        ],
    )
    def k(dst_hbm, z_hbm, o_hbm, out_hbm, didx, onesv, stage, acc):
        c = lax.axis_index("c")
        s = lax.axis_index("s")
        pltpu.sync_copy(z_hbm.at[pl.ds(0, RPT)], stage)
        pltpu.sync_copy(stage, acc.at[pl.ds(s * RPT, RPT)])

        @pl.when(s == 0)
        def _():
            pltpu.sync_copy(stage.at[pl.ds(0, TAIL)],
                            acc.at[pl.ds(TAIL_OFF, TAIL)])

        pltpu.sync_copy(o_hbm.at[pl.ds(0, K)], onesv)
        plsc.subcore_barrier()
        base = (c * NS + s) * EPT

        def body(i, carry):
            off = base + i * K
            pltpu.sync_copy(dst_hbm.at[pl.ds(off, K)], didx)
            pltpu.sync_copy(onesv, acc.at[didx], add=True)
            return carry

        lax.fori_loop(0, NCHUNK, body, 0)
        plsc.subcore_barrier()
        pltpu.sync_copy(acc.at[pl.ds(s * RPT, RPT)], stage)
        pltpu.sync_copy(stage, out_hbm.at[pl.ds(c * N + s * RPT, RPT)])

        @pl.when(s == 0)
        def _():
            pltpu.sync_copy(acc.at[pl.ds(TAIL_OFF, TAIL)],
                            onesv.at[pl.ds(0, TAIL)])
            pltpu.sync_copy(onesv.at[pl.ds(0, TAIL)],
                            out_hbm.at[pl.ds(c * N + TAIL_OFF, TAIL)])

    return k(dst, zeros1, ones1)


def _sc_scatter(hs, src, dst, zeros_rows):
    @functools.partial(
        pl.kernel,
        mesh=_mesh(),
        out_type=jax.ShapeDtypeStruct((NC, N, D), jnp.float32),
        scratch_types=[
            pltpu.VMEM((K,), jnp.int32),
            pltpu.VMEM((K,), jnp.int32),
            pltpu.VMEM((K, D), jnp.float32),
            pltpu.VMEM_SHARED((N, D), jnp.float32),
            pltpu.SemaphoreType.DMA,
        ],
    )
    def k(hs_hbm, src_hbm, dst_hbm, z_hbm, out_hbm, sidx, didx, rows, acc,
          gsem):
        c = lax.axis_index("c")
        s = lax.axis_index("s")
        pltpu.sync_copy(z_hbm, acc.at[pl.ds(s * RPT, RPT)])

        @pl.when(s == 0)
        def _():
            pltpu.sync_copy(z_hbm.at[pl.ds(0, TAIL)],
                            acc.at[pl.ds(TAIL_OFF, TAIL)])

        plsc.subcore_barrier()
        base = (c * NS + s) * EPT

        def body(i, carry):
            off = base + i * K
            pltpu.sync_copy(src_hbm.at[pl.ds(off, K)], sidx)
            pltpu.sync_copy(dst_hbm.at[pl.ds(off, K)], didx)
            pltpu.async_copy(hs_hbm.at[sidx], rows, gsem).wait()
            pltpu.sync_copy(rows, acc.at[didx], add=True)
            return carry

        lax.fori_loop(0, NCHUNK, body, 0)
        plsc.subcore_barrier()
        pltpu.sync_copy(acc.at[pl.ds(s * RPT, RPT)],
                        out_hbm.at[c, pl.ds(s * RPT, RPT)])

        @pl.when(s == 0)
        def _():
            pltpu.sync_copy(acc.at[pl.ds(TAIL_OFF, TAIL)],
                            out_hbm.at[c, pl.ds(TAIL_OFF, TAIL)])

    return k(hs, src, dst, zeros_rows)


def _tc_pre(deg2, x, W1):
    def body(deg_ref, x_ref, w_ref, dis_ref, hs_ref):
        deg = deg_ref[0] + deg_ref[1] + 1.0
        dis = lax.rsqrt(deg)
        dis_ref[...] = dis
        hw = jnp.dot(x_ref[...], w_ref[...],
                     preferred_element_type=jnp.float32)
        hs_ref[...] = dis * hw

    return pl.pallas_call(
        body,
        grid=(NBLK,),
        in_specs=[
            pl.BlockSpec((NC, BLK, 1), lambda i: (0, i, 0)),
            pl.BlockSpec((BLK, D), lambda i: (i, 0)),
            pl.BlockSpec((D, H), lambda i: (0, 0)),
        ],
        out_specs=[
            pl.BlockSpec((BLK, 1), lambda i: (i, 0)),
            pl.BlockSpec((BLK, H), lambda i: (i, 0)),
        ],
        out_shape=[
            jax.ShapeDtypeStruct((N, 1), jnp.float32),
            jax.ShapeDtypeStruct((N, H), jnp.float32),
        ],
    )(deg2, x, W1)


def _tc_mid(S2, hs, dis, prev, b, g, be, rm, rv, Wn, has_prev):
    def body(*refs):
        if has_prev:
            (s2_ref, hs_ref, dis_ref, prev_ref, b_ref, g_ref, be_ref,
             rm_ref, rv_ref, w_ref, h_ref, hsn_ref) = refs
        else:
            (s2_ref, hs_ref, dis_ref, b_ref, g_ref, be_ref,
             rm_ref, rv_ref, w_ref, h_ref, hsn_ref) = refs
        dis = dis_ref[...]
        z = dis * (s2_ref[0] + s2_ref[1] + hs_ref[...]) + b_ref[...]
        a = g_ref[...] * lax.rsqrt(rv_ref[...] + 1e-5)
        cst = be_ref[...] - rm_ref[...] * a
        h = z * a + cst
        if has_prev:
            h = h + prev_ref[...]
        h = jnp.maximum(h, 0.0)
        h_ref[...] = h
        hsn_ref[...] = dis * jnp.dot(h, w_ref[...],
                                     preferred_element_type=jnp.float32)

    in_specs = [
        pl.BlockSpec((NC, BLK, H), lambda i: (0, i, 0)),
        pl.BlockSpec((BLK, H), lambda i: (i, 0)),
        pl.BlockSpec((BLK, 1), lambda i: (i, 0)),
    ]
    args = [S2, hs, dis]
    if has_prev:
        in_specs.append(pl.BlockSpec((BLK, H), lambda i: (i, 0)))
        args.append(prev)
    in_specs += [pl.BlockSpec((1, H), lambda i: (0, 0))] * 5
    args += [b, g, be, rm, rv]
    in_specs.append(pl.BlockSpec((H, H), lambda i: (0, 0)))
    args.append(Wn)

    return pl.pallas_call(
        body,
        grid=(NBLK,),
        in_specs=in_specs,
        out_specs=[
            pl.BlockSpec((BLK, H), lambda i: (i, 0)),
            pl.BlockSpec((BLK, H), lambda i: (i, 0)),
        ],
        out_shape=[
            jax.ShapeDtypeStruct((N, H), jnp.float32),
            jax.ShapeDtypeStruct((N, H), jnp.float32),
        ],
    )(*args)


def _tc_final(S2, hs3, dis, h2, b3, g3, be3, rm3, rv3, batch2, Wo, bo):
    def body(s2_ref, hs_ref, dis_ref, prev_ref, b_ref, g_ref, be_ref,
             rm_ref, rv_ref, bat_ref, wo_ref, bo_ref, out_ref,
             s_acc, cnt_acc, mx_acc):
        i = pl.program_id(0)

        @pl.when(i == 0)
        def _():
            s_acc[...] = jnp.zeros((G, H), jnp.float32)
            cnt_acc[...] = jnp.zeros((G, H), jnp.float32)
            mx_acc[...] = jnp.full((G, H), -jnp.inf, jnp.float32)

        dis = dis_ref[...]
        z = dis * (s2_ref[0] + s2_ref[1] + hs_ref[...]) + b_ref[...]
        a = g_ref[...] * lax.rsqrt(rv_ref[...] + 1e-5)
        cst = be_ref[...] - rm_ref[...] * a
        h = jnp.maximum(z * a + cst + prev_ref[...], 0.0)

        bat = bat_ref[...]
        gids = lax.broadcasted_iota(jnp.int32, (BLK, G), 1)
        oh = (bat == gids).astype(jnp.float32)
        dn = (((0,), (0,)), ((), ()))
        s_acc[...] = s_acc[...] + lax.dot_general(
            oh, h, dn, preferred_element_type=jnp.float32)
        cnt_acc[...] = cnt_acc[...] + lax.dot_general(
            oh, jnp.ones((BLK, H), jnp.float32), dn,
            preferred_element_type=jnp.float32)

        g_lo = jnp.min(bat)
        g_hi = jnp.max(bat)

        def mbody(gg, carry):
            m = jnp.max(jnp.where(bat == gg, h, -jnp.inf), axis=0,
                        keepdims=True)
            mx_acc[pl.ds(gg, 1), :] = jnp.maximum(mx_acc[pl.ds(gg, 1), :], m)
            return carry

        lax.fori_loop(g_lo, g_hi + 1, mbody, 0)

        @pl.when(i == NBLK - 1)
        def _():
            cnt = jnp.maximum(cnt_acc[...], 1.0)
            mean = s_acc[...] / cnt
            pooled = jnp.concatenate([mean, s_acc[...], mx_acc[...]], axis=1)
            out_ref[...] = jnp.dot(pooled, wo_ref[...],
                                   preferred_element_type=jnp.float32
                                   ) + bo_ref[...]

    return pl.pallas_call(
        body,
        grid=(NBLK,),
        in_specs=[
            pl.BlockSpec((NC, BLK, H), lambda i: (0, i, 0)),
            pl.BlockSpec((BLK, H), lambda i: (i, 0)),
            pl.BlockSpec((BLK, 1), lambda i: (i, 0)),
            pl.BlockSpec((BLK, H), lambda i: (i, 0)),
        ] + [pl.BlockSpec((1, H), lambda i: (0, 0))] * 5 + [
            pl.BlockSpec((BLK, 1), lambda i: (i, 0)),
            pl.BlockSpec((3 * H, C), lambda i: (0, 0)),
            pl.BlockSpec((1, C), lambda i: (0, 0)),
        ],
        out_specs=pl.BlockSpec((G, C), lambda i: (0, 0)),
        out_shape=jax.ShapeDtypeStruct((G, C), jnp.float32),
        scratch_shapes=[
            pltpu.VMEM((G, H), jnp.float32),
            pltpu.VMEM((G, H), jnp.float32),
            pltpu.VMEM((G, H), jnp.float32),
        ],
    )(S2, hs3, dis, h2, b3, g3, be3, rm3, rv3, batch2, Wo, bo)


def kernel(x, edge_index, edge_attr, batch, W1, b1, W2, b2, W3, b3,
           g1, be1, rm1, rv1, g2, be2, rm2, rv2, g3, be3, rm3, rv3, Wo, bo):
    del edge_attr
    zeros1 = jnp.zeros((RPT + TAIL,), jnp.float32)
    ones1 = jnp.ones((K + 8,), jnp.float32)
    zrows = jnp.zeros((RPT, D), jnp.float32)
    r2 = lambda v: v.reshape(1, -1)
    batch2 = batch.reshape(N, 1)
    src = edge_index[0]
    dst = edge_index[1]

    deg2 = _sc_degree(dst, zeros1, ones1).reshape(NC, N, 1)
    dis, hs1 = _tc_pre(deg2, x, W1)
    S1 = _sc_scatter(hs1, src, dst, zrows)
    h1, hs2 = _tc_mid(S1, hs1, dis, None, r2(b1), r2(g1), r2(be1),
                      r2(rm1), r2(rv1), W2, has_prev=False)
    S2 = _sc_scatter(hs2, src, dst, zrows)
    h2, hs3 = _tc_mid(S2, hs2, dis, h1, r2(b2), r2(g2), r2(be2),
                      r2(rm2), r2(rv2), W3, has_prev=True)
    S3 = _sc_scatter(hs3, src, dst, zrows)
    out = _tc_final(S3, hs3, dis, h2, r2(b3), r2(g3), r2(be3),
                    r2(rm3), r2(rv3), batch2, Wo, r2(bo))
    return out
